# baseline (speedup 1.0000x reference)
import functools

import jax
import jax.numpy as jnp
from jax import lax
from jax.experimental import pallas as pl
from jax.experimental.pallas import tpu as pltpu
from jax.experimental.pallas import tpu_sc as plsc

N = 50000
D = 256
D2 = D // 2
TBL = 64
L = 16
NC = 2
NS = 16
NW = NC * NS
R = 80
NBLK = N // R
KMAIN = 19
MAIN = NW * KMAIN
TAIL = NBLK - MAIN

_mesh = plsc.VectorSubcoreMesh(core_axis_name="c", subcore_axis_name="s")

_GATHER_DNUMS = lax.GatherDimensionNumbers(
    offset_dims=(), collapsed_slice_dims=(0,), start_index_map=(0,))


@functools.partial(
    pl.kernel,
    mesh=_mesh,
    compiler_params=pltpu.CompilerParams(needs_layout_passes=False),
    out_type=jax.ShapeDtypeStruct((N, D), jnp.float32),
    scratch_types=[
        pltpu.VMEM((KMAIN * R,), jnp.int32),
        pltpu.VMEM((R,), jnp.int32),
        pltpu.VMEM((TBL * D2,), jnp.float32),
        pltpu.VMEM((TBL, D), jnp.float32),
        pltpu.VMEM((TBL, D), jnp.float32),
        pltpu.VMEM((R, D), jnp.float32),
        pltpu.VMEM((R, D), jnp.float32),
        pltpu.SemaphoreType.DMA,
        pltpu.SemaphoreType.DMA,
        pltpu.SemaphoreType.DMA,
        pltpu.SemaphoreType.DMA,
        pltpu.SemaphoreType.DMA,
    ],
)
def _sc_add(x_hbm, rank_hbm, zin_hbm, zout_hbm, out_hbm,
            idx_all, idx_t, zc, tz0, tz1, xb0, xb1,
            sem_i, sem_x0, sem_x1, sem_o0, sem_o1):
    cid = lax.axis_index("c")
    sid = lax.axis_index("s")
    wid = sid * NC + cid

    xbufs = (xb0, xb1)
    semx = (sem_x0, sem_x1)
    semo = (sem_o0, sem_o1)

    s0 = wid * KMAIN
    pltpu.async_copy(rank_hbm.at[pl.ds(s0 * R, KMAIN * R)], idx_all, sem_i)
    pltpu.async_copy(x_hbm.at[pl.ds(s0 * R, R)], xb0, sem_x0)

    pltpu.sync_copy(zin_hbm, tz0)
    pltpu.sync_copy(zout_hbm, tz1)

    def trow(r, c2):
        for c8 in range(D // (2 * L)):
            sa = pl.ds(c8 * 2 * L, L)
            sb = pl.ds(c8 * 2 * L + L, L)
            a = tz0[r, sa] + tz1[r, sa]
            b = tz0[r, sb] + tz1[r, sb]
            p = plsc.pack(a, b, format=plsc.PackFormat.INTERLEAVED)
            zc[pl.ds(r * D2 + c8 * L, L)] = plsc.bitcast(p, jnp.float32)
        return c2

    lax.fori_loop(0, TBL, trow, 0)
    pltpu.make_async_copy(rank_hbm.at[pl.ds(s0 * R, KMAIN * R)], idx_all,
                          sem_i).wait()

    lane = lax.iota(jnp.int32, L)

    def fire_x(k, slot):
        pltpu.async_copy(x_hbm.at[pl.ds((s0 + k) * R, R)], xbufs[slot],
                         semx[slot])

    def wait_x(k, slot):
        pltpu.make_async_copy(x_hbm.at[pl.ds((s0 + k) * R, R)], xbufs[slot],
                              semx[slot]).wait()

    def fire_out(k, slot):
        pltpu.async_copy(xbufs[slot], out_hbm.at[pl.ds((s0 + k) * R, R)],
                         semo[slot])

    def wait_out(k, slot):
        pltpu.make_async_copy(xbufs[slot], out_hbm.at[pl.ds((s0 + k) * R, R)],
                              semo[slot]).wait()

    def _lane_broadcast(vec, l):
        idx = (lane * 0 + l)[:, None]
        return lax.gather(vec, idx, _GATHER_DNUMS, slice_sizes=(1,),
                          mode=lax.GatherScatterMode.PROMISE_IN_BOUNDS)

    def add_block(xb, idx_ref, ibase):
        def grp(j, c2):
            rv = idx_ref[pl.ds(ibase + j * L, L)]
            for l in range(L):
                ri = _lane_broadcast(rv, l)
                zrow = ri * D2 + lane
                i = j * L + l
                for c in range(D2 // L):
                    v = plsc.load_gather(zc, [zrow + c * L])
                    vb = plsc.bitcast(v, jnp.bfloat16)
                    a, b = plsc.unpack(vb, format=plsc.PackFormat.INTERLEAVED)
                    sa = pl.ds(c * 2 * L, L)
                    sb = pl.ds(c * 2 * L + L, L)
                    xb[i, sa] = xb[i, sa] + a
                    xb[i, sb] = xb[i, sb] + b
            return c2

        lax.fori_loop(0, R // L, grp, 0)

    def pair_body(k2, carry):
        for u in (0, 1):
            s, sp = u, 1 - u
            k = k2 * 2 + u

            @pl.when((k >= 1) & (k <= KMAIN))
            def _drain_prev():
                wait_out(k - 1, sp)

            @pl.when(k + 1 < KMAIN)
            def _prefetch():
                fire_x(k + 1, sp)

            @pl.when(k < KMAIN)
            def _process():
                wait_x(k, s)
                add_block(xbufs[s], idx_all, k * R)
                fire_out(k, s)

        return carry

    lax.fori_loop(0, (KMAIN + 2) // 2, pair_body, 0)

    @pl.when(wid < TAIL)
    def _tail():
        tb = MAIN + wid
        pltpu.sync_copy(rank_hbm.at[pl.ds(tb * R, R)], idx_t)
        pltpu.async_copy(x_hbm.at[pl.ds(tb * R, R)], xb1, sem_x1)
        pltpu.make_async_copy(x_hbm.at[pl.ds(tb * R, R)], xb1, sem_x1).wait()
        add_block(xb1, idx_t, 0)
        pltpu.sync_copy(xb1, out_hbm.at[pl.ds(tb * R, R)])


def kernel(x, rank, z_in, z_out):
    return _sc_add(x, rank.astype(jnp.int32), z_in, z_out)


# speedup vs baseline: 1.2901x; 1.2901x over previous
import functools

import jax
import jax.numpy as jnp
from jax import lax
from jax.experimental import pallas as pl
from jax.experimental.pallas import tpu as pltpu
from jax.experimental.pallas import tpu_sc as plsc

N = 50000
D = 256
D2 = D // 2
TBL = 64
L = 16
NC = 2
NS = 16
NW = NC * NS
R = 80
NBLK = N // R
KMAIN = 19
MAIN = NW * KMAIN
TAIL = NBLK - MAIN

_mesh = plsc.VectorSubcoreMesh(core_axis_name="c", subcore_axis_name="s")

_GATHER_DNUMS = lax.GatherDimensionNumbers(
    offset_dims=(), collapsed_slice_dims=(0,), start_index_map=(0,))


def _combine_tables(z_in, z_out):
    def body(a_ref, b_ref, o_ref):
        o_ref[...] = (a_ref[...] + b_ref[...]).astype(jnp.bfloat16)

    return pl.pallas_call(
        body,
        out_shape=jax.ShapeDtypeStruct((TBL, D), jnp.bfloat16),
    )(z_in, z_out)


def _pack_table(z16):
    t = z16.reshape(TBL, D // 32, 2, L).transpose(0, 1, 3, 2)
    return lax.bitcast_convert_type(t, jnp.float32).reshape(TBL * D2)


@functools.partial(
    pl.kernel,
    mesh=_mesh,
    compiler_params=pltpu.CompilerParams(needs_layout_passes=False),
    out_type=jax.ShapeDtypeStruct((N, D), jnp.float32),
    scratch_types=[
        pltpu.VMEM((KMAIN * R,), jnp.int32),
        pltpu.VMEM((R,), jnp.int32),
        pltpu.VMEM((TBL * D2,), jnp.float32),
        pltpu.VMEM((R, D), jnp.float32),
        pltpu.VMEM((R, D), jnp.float32),
        pltpu.SemaphoreType.DMA,
        pltpu.SemaphoreType.DMA,
        pltpu.SemaphoreType.DMA,
        pltpu.SemaphoreType.DMA,
        pltpu.SemaphoreType.DMA,
    ],
)
def _sc_add(x_hbm, rank_hbm, zc_hbm, out_hbm,
            idx_all, idx_t, zc, xb0, xb1,
            sem_i, sem_x0, sem_x1, sem_o0, sem_o1):
    cid = lax.axis_index("c")
    sid = lax.axis_index("s")
    wid = sid * NC + cid

    xbufs = (xb0, xb1)
    semx = (sem_x0, sem_x1)
    semo = (sem_o0, sem_o1)

    s0 = wid * KMAIN
    pltpu.async_copy(rank_hbm.at[pl.ds(s0 * R, KMAIN * R)], idx_all, sem_i)
    pltpu.async_copy(x_hbm.at[pl.ds(s0 * R, R)], xb0, sem_x0)
    pltpu.sync_copy(zc_hbm, zc)
    pltpu.make_async_copy(rank_hbm.at[pl.ds(s0 * R, KMAIN * R)], idx_all,
                          sem_i).wait()

    lane = lax.iota(jnp.int32, L)

    def fire_x(k, slot):
        pltpu.async_copy(x_hbm.at[pl.ds((s0 + k) * R, R)], xbufs[slot],
                         semx[slot])

    def wait_x(k, slot):
        pltpu.make_async_copy(x_hbm.at[pl.ds((s0 + k) * R, R)], xbufs[slot],
                              semx[slot]).wait()

    def fire_out(k, slot):
        pltpu.async_copy(xbufs[slot], out_hbm.at[pl.ds((s0 + k) * R, R)],
                         semo[slot])

    def wait_out(k, slot):
        pltpu.make_async_copy(xbufs[slot], out_hbm.at[pl.ds((s0 + k) * R, R)],
                              semo[slot]).wait()

    def _lane_broadcast(vec, l):
        idx = (lane * 0 + l)[:, None]
        return lax.gather(vec, idx, _GATHER_DNUMS, slice_sizes=(1,),
                          mode=lax.GatherScatterMode.PROMISE_IN_BOUNDS)

    def add_block(xb, idx_ref, ibase):
        def grp(j, c2):
            rv = idx_ref[pl.ds(ibase + j * L, L)]
            for l in range(L):
                ri = _lane_broadcast(rv, l)
                zrow = ri * D2 + lane
                i = j * L + l
                for c in range(D2 // L):
                    v = plsc.load_gather(zc, [zrow + c * L])
                    vb = plsc.bitcast(v, jnp.bfloat16)
                    a, b = plsc.unpack(vb, format=plsc.PackFormat.INTERLEAVED)
                    sa = pl.ds(c * 2 * L, L)
                    sb = pl.ds(c * 2 * L + L, L)
                    plsc.addupdate(xb.at[i, sa], a)
                    plsc.addupdate(xb.at[i, sb], b)
            return c2

        lax.fori_loop(0, R // L, grp, 0)

    def pair_body(k2, carry):
        for u in (0, 1):
            s, sp = u, 1 - u
            k = k2 * 2 + u

            @pl.when((k >= 1) & (k <= KMAIN))
            def _drain_prev():
                wait_out(k - 1, sp)

            @pl.when(k + 1 < KMAIN)
            def _prefetch():
                fire_x(k + 1, sp)

            @pl.when(k < KMAIN)
            def _process():
                wait_x(k, s)
                add_block(xbufs[s], idx_all, k * R)
                fire_out(k, s)

        return carry

    lax.fori_loop(0, (KMAIN + 2) // 2, pair_body, 0)

    @pl.when(wid < TAIL)
    def _tail():
        tb = MAIN + wid
        pltpu.sync_copy(rank_hbm.at[pl.ds(tb * R, R)], idx_t)
        pltpu.async_copy(x_hbm.at[pl.ds(tb * R, R)], xb1, sem_x1)
        pltpu.make_async_copy(x_hbm.at[pl.ds(tb * R, R)], xb1, sem_x1).wait()
        add_block(xb1, idx_t, 0)
        pltpu.sync_copy(xb1, out_hbm.at[pl.ds(tb * R, R)])


def kernel(x, rank, z_in, z_out):
    zc = _pack_table(_combine_tables(z_in, z_out))
    return _sc_add(x, rank.astype(jnp.int32), zc)


# speedup vs baseline: 1.2919x; 1.0014x over previous
import functools

import jax
import jax.numpy as jnp
from jax import lax
from jax.experimental import pallas as pl
from jax.experimental.pallas import tpu as pltpu
from jax.experimental.pallas import tpu_sc as plsc

N = 50000
D = 256
D2 = D // 2
TBL = 64
L = 16
NC = 2
NS = 16
NW = NC * NS
R = 80
NBLK = N // R
KMAIN = 19
MAIN = NW * KMAIN
TAIL = NBLK - MAIN

_mesh = plsc.VectorSubcoreMesh(core_axis_name="c", subcore_axis_name="s")

_GATHER_DNUMS = lax.GatherDimensionNumbers(
    offset_dims=(), collapsed_slice_dims=(0,), start_index_map=(0,))


def _combine_tables(z_in, z_out):
    def body(a_ref, b_ref, o_ref):
        s = (a_ref[...] + b_ref[...]).astype(jnp.bfloat16)
        u = lax.bitcast_convert_type(s, jnp.uint16).astype(jnp.uint32)
        lo = jnp.concatenate(
            [u[:, c8 * 32:c8 * 32 + L] for c8 in range(D // 32)], axis=1)
        hi = jnp.concatenate(
            [u[:, c8 * 32 + L:c8 * 32 + 2 * L] for c8 in range(D // 32)],
            axis=1)
        o_ref[...] = lax.bitcast_convert_type(lo | (hi << 16), jnp.float32)

    return pl.pallas_call(
        body,
        out_shape=jax.ShapeDtypeStruct((TBL, D2), jnp.float32),
    )(z_in, z_out)


@functools.partial(
    pl.kernel,
    mesh=_mesh,
    compiler_params=pltpu.CompilerParams(needs_layout_passes=False),
    out_type=jax.ShapeDtypeStruct((N, D), jnp.float32),
    scratch_types=[
        pltpu.VMEM((KMAIN * R,), jnp.int32),
        pltpu.VMEM((R,), jnp.int32),
        pltpu.VMEM((TBL * D2,), jnp.float32),
        pltpu.VMEM((R, D), jnp.float32),
        pltpu.VMEM((R, D), jnp.float32),
        pltpu.SemaphoreType.DMA,
        pltpu.SemaphoreType.DMA,
        pltpu.SemaphoreType.DMA,
        pltpu.SemaphoreType.DMA,
        pltpu.SemaphoreType.DMA,
    ],
)
def _sc_add(x_hbm, rank_hbm, zc_hbm, out_hbm,
            idx_all, idx_t, zc, xb0, xb1,
            sem_i, sem_x0, sem_x1, sem_o0, sem_o1):
    cid = lax.axis_index("c")
    sid = lax.axis_index("s")
    wid = sid * NC + cid

    xbufs = (xb0, xb1)
    semx = (sem_x0, sem_x1)
    semo = (sem_o0, sem_o1)

    s0 = wid * KMAIN
    pltpu.async_copy(rank_hbm.at[pl.ds(s0 * R, KMAIN * R)], idx_all, sem_i)
    pltpu.async_copy(x_hbm.at[pl.ds(s0 * R, R)], xb0, sem_x0)
    pltpu.sync_copy(zc_hbm, zc)
    pltpu.make_async_copy(rank_hbm.at[pl.ds(s0 * R, KMAIN * R)], idx_all,
                          sem_i).wait()

    lane = lax.iota(jnp.int32, L)

    def fire_x(k, slot):
        pltpu.async_copy(x_hbm.at[pl.ds((s0 + k) * R, R)], xbufs[slot],
                         semx[slot])

    def wait_x(k, slot):
        pltpu.make_async_copy(x_hbm.at[pl.ds((s0 + k) * R, R)], xbufs[slot],
                              semx[slot]).wait()

    def fire_out(k, slot):
        pltpu.async_copy(xbufs[slot], out_hbm.at[pl.ds((s0 + k) * R, R)],
                         semo[slot])

    def wait_out(k, slot):
        pltpu.make_async_copy(xbufs[slot], out_hbm.at[pl.ds((s0 + k) * R, R)],
                              semo[slot]).wait()

    def _lane_broadcast(vec, l):
        idx = (lane * 0 + l)[:, None]
        return lax.gather(vec, idx, _GATHER_DNUMS, slice_sizes=(1,),
                          mode=lax.GatherScatterMode.PROMISE_IN_BOUNDS)

    def add_block(xb, idx_ref, ibase):
        def grp(j, c2):
            rv = idx_ref[pl.ds(ibase + j * L, L)]
            for l in range(L):
                ri = _lane_broadcast(rv, l)
                zrow = ri * D2 + lane
                i = j * L + l
                for c in range(D2 // L):
                    v = plsc.load_gather(zc, [zrow + c * L])
                    vb = plsc.bitcast(v, jnp.bfloat16)
                    a, b = plsc.unpack(vb, format=plsc.PackFormat.INTERLEAVED)
                    sa = pl.ds(c * 2 * L, L)
                    sb = pl.ds(c * 2 * L + L, L)
                    plsc.addupdate(xb.at[i, sa], a)
                    plsc.addupdate(xb.at[i, sb], b)
            return c2

        lax.fori_loop(0, R // L, grp, 0)

    def pair_body(k2, carry):
        for u in (0, 1):
            s, sp = u, 1 - u
            k = k2 * 2 + u

            @pl.when((k >= 1) & (k <= KMAIN))
            def _drain_prev():
                wait_out(k - 1, sp)

            @pl.when(k + 1 < KMAIN)
            def _prefetch():
                fire_x(k + 1, sp)

            @pl.when(k < KMAIN)
            def _process():
                wait_x(k, s)
                add_block(xbufs[s], idx_all, k * R)
                fire_out(k, s)

        return carry

    lax.fori_loop(0, (KMAIN + 2) // 2, pair_body, 0)

    @pl.when(wid < TAIL)
    def _tail():
        tb = MAIN + wid
        pltpu.sync_copy(rank_hbm.at[pl.ds(tb * R, R)], idx_t)
        pltpu.async_copy(x_hbm.at[pl.ds(tb * R, R)], xb1, sem_x1)
        pltpu.make_async_copy(x_hbm.at[pl.ds(tb * R, R)], xb1, sem_x1).wait()
        add_block(xb1, idx_t, 0)
        pltpu.sync_copy(xb1, out_hbm.at[pl.ds(tb * R, R)])


def kernel(x, rank, z_in, z_out):
    zc = _combine_tables(z_in, z_out).reshape(TBL * D2)
    return _sc_add(x, rank.astype(jnp.int32), zc)


# speedup vs baseline: 1.3364x; 1.0344x over previous
import functools

import jax
import jax.numpy as jnp
from jax import lax
from jax.experimental import pallas as pl
from jax.experimental.pallas import tpu as pltpu
from jax.experimental.pallas import tpu_sc as plsc

N = 50000
D = 256
D2 = D // 2
TBL = 64
L = 16
NC = 2
NS = 16
NW = NC * NS
R = 80
NBLK = N // R
KMAX = 20
NLONG = NBLK - NW * (KMAX - 1)

_mesh = plsc.VectorSubcoreMesh(core_axis_name="c", subcore_axis_name="s")

_GATHER_DNUMS = lax.GatherDimensionNumbers(
    offset_dims=(), collapsed_slice_dims=(0,), start_index_map=(0,))


def _combine_tables(z_in, z_out):
    def body(a_ref, b_ref, o_ref):
        s = (a_ref[...] + b_ref[...]).astype(jnp.bfloat16)
        u = lax.bitcast_convert_type(s, jnp.uint16).astype(jnp.uint32)
        lo = jnp.concatenate(
            [u[:, c8 * 32:c8 * 32 + L] for c8 in range(D // 32)], axis=1)
        hi = jnp.concatenate(
            [u[:, c8 * 32 + L:c8 * 32 + 2 * L] for c8 in range(D // 32)],
            axis=1)
        o_ref[...] = lax.bitcast_convert_type(lo | (hi << 16), jnp.float32)

    return pl.pallas_call(
        body,
        out_shape=jax.ShapeDtypeStruct((TBL, D2), jnp.float32),
    )(z_in, z_out)


@functools.partial(
    pl.kernel,
    mesh=_mesh,
    compiler_params=pltpu.CompilerParams(needs_layout_passes=False),
    out_type=jax.ShapeDtypeStruct((N, D), jnp.float32),
    scratch_types=[
        pltpu.VMEM((KMAX * R,), jnp.int32),
        pltpu.VMEM((TBL * D2,), jnp.float32),
        pltpu.VMEM((R, D), jnp.float32),
        pltpu.VMEM((R, D), jnp.float32),
        pltpu.SemaphoreType.DMA,
        pltpu.SemaphoreType.DMA,
        pltpu.SemaphoreType.DMA,
        pltpu.SemaphoreType.DMA,
        pltpu.SemaphoreType.DMA,
    ],
)
def _sc_add(x_hbm, rank_hbm, zc_hbm, out_hbm,
            idx_all, zc, xb0, xb1,
            sem_i, sem_x0, sem_x1, sem_o0, sem_o1):
    cid = lax.axis_index("c")
    sid = lax.axis_index("s")
    wid = sid * NC + cid

    xbufs = (xb0, xb1)
    semx = (sem_x0, sem_x1)
    semo = (sem_o0, sem_o1)

    long_w = wid < NLONG
    cnt = jnp.where(long_w, KMAX, KMAX - 1)
    s0 = wid * KMAX - jnp.maximum(wid - NLONG, 0)
    off = jnp.where(long_w, 0, 1)
    cstart = s0 - off
    pltpu.async_copy(rank_hbm.at[pl.ds(cstart * R, KMAX * R)], idx_all, sem_i)
    pltpu.async_copy(x_hbm.at[pl.ds(s0 * R, R)], xb0, sem_x0)
    pltpu.sync_copy(zc_hbm, zc)
    pltpu.make_async_copy(rank_hbm.at[pl.ds(cstart * R, KMAX * R)], idx_all,
                          sem_i).wait()

    lane = lax.iota(jnp.int32, L)

    def fire_x(k, slot):
        pltpu.async_copy(x_hbm.at[pl.ds((s0 + k) * R, R)], xbufs[slot],
                         semx[slot])

    def wait_x(k, slot):
        pltpu.make_async_copy(x_hbm.at[pl.ds((s0 + k) * R, R)], xbufs[slot],
                              semx[slot]).wait()

    def fire_out(k, slot):
        pltpu.async_copy(xbufs[slot], out_hbm.at[pl.ds((s0 + k) * R, R)],
                         semo[slot])

    def wait_out(k, slot):
        pltpu.make_async_copy(xbufs[slot], out_hbm.at[pl.ds((s0 + k) * R, R)],
                              semo[slot]).wait()

    def _lane_broadcast(vec, l):
        idx = (lane * 0 + l)[:, None]
        return lax.gather(vec, idx, _GATHER_DNUMS, slice_sizes=(1,),
                          mode=lax.GatherScatterMode.PROMISE_IN_BOUNDS)

    def add_block(xb, idx_ref, ibase):
        def grp(j, c2):
            rv = idx_ref[pl.ds(ibase + j * L, L)]
            for l in range(L):
                ri = _lane_broadcast(rv, l)
                zrow = ri * D2 + lane
                i = j * L + l
                for c in range(D2 // L):
                    v = plsc.load_gather(zc, [zrow + c * L])
                    vb = plsc.bitcast(v, jnp.bfloat16)
                    a, b = plsc.unpack(vb, format=plsc.PackFormat.INTERLEAVED)
                    sa = pl.ds(c * 2 * L, L)
                    sb = pl.ds(c * 2 * L + L, L)
                    plsc.addupdate(xb.at[i, sa], a)
                    plsc.addupdate(xb.at[i, sb], b)
            return c2

        lax.fori_loop(0, R // L, grp, 0)

    def pair_body(k2, carry):
        for u in (0, 1):
            s, sp = u, 1 - u
            k = k2 * 2 + u

            @pl.when((k >= 1) & (k - 1 < cnt))
            def _drain_prev():
                wait_out(k - 1, sp)

            @pl.when(k + 1 < cnt)
            def _prefetch():
                fire_x(k + 1, sp)

            @pl.when(k < cnt)
            def _process():
                wait_x(k, s)
                add_block(xbufs[s], idx_all, (k + off) * R)
                fire_out(k, s)

        return carry

    lax.fori_loop(0, KMAX // 2, pair_body, 0)

    @pl.when(long_w)
    def _drain_last():
        wait_out(KMAX - 1, (KMAX - 1) % 2)


def kernel(x, rank, z_in, z_out):
    zc = _combine_tables(z_in, z_out).reshape(TBL * D2)
    return _sc_add(x, rank.astype(jnp.int32), zc)
